# Initial kernel scaffold; baseline (speedup 1.0000x reference)
import jax
import jax.numpy as jnp
from jax.experimental import pallas as pl


def kernel(tokens, token_types, t, W1, b1, W2, b2):
    raise NotImplementedError("write your pallas kernel here")


# speedup vs baseline: 1.7341x; 1.7341x over previous
import functools

import jax
import jax.numpy as jnp
from jax.experimental import pallas as pl
from jax.experimental.pallas import tpu as pltpu

E = 8
TEMP = 0.1
ALPHA = 0.3
UNIF = ALPHA / E
KNOWN_V = 1.0 - ALPHA + UNIF
TTYPE_UNKNOWN = 5
INV7 = 1.0 / 7.0


def _gelu_exact(x):
    return 0.5 * x * (1.0 + jax.lax.erf(x * 0.7071067811865476))


def _dense_body(types_ref, t_ref, x_ref, w1_ref, b1_ref, w2_ref, b2_ref,
                disp_ref, comb_ref, *, blk, blocks_per_batch):
    i = pl.program_id(0)
    b = i // blocks_per_batch
    cap = 0.5 + 1.0e-4 * t_ref[b].astype(jnp.float32)

    x = x_ref[...]
    h = jnp.dot(x, w1_ref[...], preferred_element_type=jnp.float32)
    h = _gelu_exact(h + b1_ref[...])
    logits = (jnp.dot(h, w2_ref[...], preferred_element_type=jnp.float32)
              + b2_ref[...]) * TEMP

    m = jnp.max(logits, axis=-1, keepdims=True)
    sumexp = jnp.sum(jnp.exp(logits - m), axis=-1, keepdims=True)
    e_iota = jax.lax.broadcasted_iota(jnp.int32, (blk, E), 1)
    j_soft = jnp.min(jnp.where(logits >= m, e_iota, E), axis=-1, keepdims=True)
    v_soft = (1.0 - ALPHA) / sumexp + UNIF

    types = types_ref[...]
    is_unk = types == TTYPE_UNKNOWN
    j = jnp.where(is_unk, j_soft, types)
    v = jnp.where(is_unk, v_soft, KNOWN_V)

    onehot = e_iota == j
    d = jnp.where(onehot, jnp.minimum(v, cap),
                  jnp.maximum(v - cap, 0.0) * INV7)
    sumd = jnp.sum(d, axis=-1, keepdims=True)
    disp_ref[...] = d
    comb_ref[...] = d / (sumd + 1e-8)


def kernel(tokens, token_types, t, W1, b1, W2, b2):
    B, N, D = tokens.shape
    H = W1.shape[1]
    T = B * N
    BLK = 2048
    nb = T // BLK
    bpb = N // BLK

    x = tokens.reshape(T, D)
    types = token_types.reshape(T, 1).astype(jnp.int32)

    body = functools.partial(_dense_body, blk=BLK, blocks_per_batch=bpb)
    disp, comb = pl.pallas_call(
        body,
        grid=(nb,),
        in_specs=[
            pl.BlockSpec((BLK, 1), lambda i: (i, 0)),
            pl.BlockSpec(memory_space=pltpu.SMEM),
            pl.BlockSpec((BLK, D), lambda i: (i, 0)),
            pl.BlockSpec((D, H), lambda i: (0, 0)),
            pl.BlockSpec((H,), lambda i: (0,)),
            pl.BlockSpec((H, E), lambda i: (0, 0)),
            pl.BlockSpec((E,), lambda i: (0,)),
        ],
        out_specs=[
            pl.BlockSpec((BLK, E), lambda i: (i, 0)),
            pl.BlockSpec((BLK, E), lambda i: (i, 0)),
        ],
        out_shape=[
            jax.ShapeDtypeStruct((T, E), jnp.float32),
            jax.ShapeDtypeStruct((T, E), jnp.float32),
        ],
    )(types, t.astype(jnp.int32), x, W1, b1, W2, b2)

    return disp.reshape(B, N, E), comb.reshape(B, N, E)
